# baseline (speedup 1.0000x reference)
import jax
import jax.numpy as jnp
from jax.experimental import pallas as pl


_BLOCK = 10000


def _body(x_ref, o_ref):
    o_ref[...] = x_ref[...]

    @pl.when(pl.program_id(0) == 0)
    def _():
        o_ref[0:1, :] = jnp.ones((1, o_ref.shape[1]), o_ref.dtype)


def kernel(x):
    n, d = x.shape
    xw = x.reshape(n // 2, d * 2)
    out = pl.pallas_call(
        _body,
        grid=(xw.shape[0] // _BLOCK,),
        in_specs=[pl.BlockSpec((_BLOCK, xw.shape[1]), lambda i: (i, 0))],
        out_specs=pl.BlockSpec((_BLOCK, xw.shape[1]), lambda i: (i, 0)),
        out_shape=jax.ShapeDtypeStruct(xw.shape, x.dtype),
    )(xw)
    return out.reshape(n, d)
